# baseline (speedup 1.0000x reference)
import functools

import jax
import jax.numpy as jnp
import numpy as np
from jax.experimental import pallas as pl
from jax.experimental.pallas import tpu as pltpu

_EPS = 1e-08


def _block(n: int, target: int) -> int:
    for b in range(min(n, target), 7, -1):
        if n % b == 0 and b % 8 == 0:
            return b
    return n


def _fwd_body(bm, adj_ref, x_ref, w_ref, d_ref, g_ref, y1_ref, gs_ref):
    i = pl.program_id(0)

    @pl.when(i == 0)
    def _init():
        gs_ref[...] = jnp.zeros_like(gs_ref)

    a = adj_ref[...]
    deg = jnp.sum(a, axis=1, keepdims=True) + 1.0
    dis = jax.lax.rsqrt(jnp.maximum(deg, _EPS))
    d_ref[...] = dis
    h = jax.lax.dot_general(
        x_ref[...], w_ref[...], (((1,), (1,)), ((), ())),
        preferred_element_type=jnp.float32)
    g = dis * h
    g_ref[...] = g
    gs_ref[pl.ds(i * bm, bm), :] = g
    y1_ref[...] = g + jnp.dot(a, gs_ref[...],
                              preferred_element_type=jnp.float32)


def _upper_body(bm, ck, nkc, n,
                si, sk, sf, sl, adj_ref, g_ref, y1_ref, d_ref, o_ref):
    t = pl.program_id(0)
    i = si[t]
    k = sk[t]
    col0 = k * ck
    start = (i + 1) * bm

    rows = col0 + jax.lax.broadcasted_iota(jnp.int32, (ck, o_ref.shape[1]), 0)
    gc = g_ref[pl.ds(col0, ck), :]
    gc = jnp.where((rows >= start) & (rows < n), gc, 0.0)

    def dot_plain():
        return jnp.dot(adj_ref[...], gc, preferred_element_type=jnp.float32)

    def dot_masked():
        cols = col0 + jax.lax.broadcasted_iota(jnp.int32, adj_ref.shape, 1)
        a = jnp.where(cols < n, adj_ref[...], 0.0)
        return jnp.dot(a, gc, preferred_element_type=jnp.float32)

    contrib = jax.lax.cond(k == nkc - 1, dot_masked, dot_plain)

    @pl.when(sf[t] == 1)
    def _first():
        o_ref[...] = contrib

    @pl.when(sf[t] == 0)
    def _accum():
        o_ref[...] += contrib

    @pl.when(sl[t] == 1)
    def _epilogue():
        o_ref[...] = jnp.maximum(d_ref[...] * (o_ref[...] + y1_ref[...]), 0.0)


def kernel(x, adj, W):
    n, f_in = x.shape
    f_out = W.shape[0]

    bm = _block(n, 400)
    ni = n // bm
    ck = min(1280, ((n + 127) // 128) * 128)
    nkc = -(-n // ck)
    gpad = nkc * ck

    d, g, y1 = pl.pallas_call(
        functools.partial(_fwd_body, bm),
        grid=(ni,),
        in_specs=[
            pl.BlockSpec((bm, n), lambda i: (i, 0)),
            pl.BlockSpec((bm, f_in), lambda i: (i, 0)),
            pl.BlockSpec((f_out, f_in), lambda i: (0, 0)),
        ],
        out_specs=[
            pl.BlockSpec((bm, 1), lambda i: (i, 0)),
            pl.BlockSpec((bm, f_out), lambda i: (i, 0)),
            pl.BlockSpec((bm, f_out), lambda i: (i, 0)),
        ],
        out_shape=[
            jax.ShapeDtypeStruct((n, 1), jnp.float32),
            jax.ShapeDtypeStruct((gpad, f_out), jnp.float32),
            jax.ShapeDtypeStruct((n, f_out), jnp.float32),
        ],
        scratch_shapes=[pltpu.VMEM((n, f_out), jnp.float32)],
    )(adj, x, W)

    i_l, k_l, f_l, l_l = [], [], [], []
    for i in range(ni):
        kmin = min((bm * (i + 1)) // ck, nkc - 1)
        for k in range(kmin, nkc):
            i_l.append(i)
            k_l.append(k)
            f_l.append(1 if k == kmin else 0)
            l_l.append(1 if k == nkc - 1 else 0)

    grid_spec = pltpu.PrefetchScalarGridSpec(
        num_scalar_prefetch=4,
        grid=(len(i_l),),
        in_specs=[
            pl.BlockSpec((bm, ck), lambda t, si, sk, sf, sl: (si[t], sk[t])),
            pl.BlockSpec((gpad, f_out), lambda t, si, sk, sf, sl: (0, 0)),
            pl.BlockSpec((bm, f_out), lambda t, si, sk, sf, sl: (si[t], 0)),
            pl.BlockSpec((bm, 1), lambda t, si, sk, sf, sl: (si[t], 0)),
        ],
        out_specs=pl.BlockSpec(
            (bm, f_out), lambda t, si, sk, sf, sl: (si[t], 0)),
    )

    out = pl.pallas_call(
        functools.partial(_upper_body, bm, ck, nkc, n),
        grid_spec=grid_spec,
        out_shape=jax.ShapeDtypeStruct((n, f_out), jnp.float32),
    )(jnp.asarray(np.asarray(i_l), jnp.int32),
      jnp.asarray(np.asarray(k_l), jnp.int32),
      jnp.asarray(np.asarray(f_l), jnp.int32),
      jnp.asarray(np.asarray(l_l), jnp.int32),
      adj, g, y1, d)

    return out


# speedup vs baseline: 1.0179x; 1.0179x over previous
import functools

import jax
import jax.numpy as jnp
import numpy as np
from jax.experimental import pallas as pl
from jax.experimental.pallas import tpu as pltpu

_EPS = 1e-08


def _block(n: int, target: int) -> int:
    for b in range(min(n, target), 7, -1):
        if n % b == 0 and b % 8 == 0:
            return b
    return n


def _fwd_body(bm, adj_ref, x_ref, w_ref, d_ref, g_ref, y1_ref, gs_ref):
    i = pl.program_id(0)

    @pl.when(i == 0)
    def _init():
        gs_ref[...] = jnp.zeros_like(gs_ref)

    a = adj_ref[...]
    deg = jnp.sum(a, axis=1, keepdims=True) + 1.0
    dis = jax.lax.rsqrt(jnp.maximum(deg, _EPS))
    d_ref[...] = dis
    h = jax.lax.dot_general(
        x_ref[...], w_ref[...], (((1,), (1,)), ((), ())),
        preferred_element_type=jnp.float32)
    g = dis * h
    g_ref[...] = g
    gs_ref[pl.ds(i * bm, bm), :] = g
    y1_ref[...] = g + jnp.dot(a, gs_ref[...],
                              preferred_element_type=jnp.float32)


def _upper_body(bm, ck, nkc, n,
                si, sk, sf, sl, adj_ref, g_ref, y1_ref, d_ref, o_ref):
    t = pl.program_id(0)
    i = si[t]
    k = sk[t]
    col0 = k * ck
    start = (i + 1) * bm

    def dot_plain():
        return jnp.dot(adj_ref[...], g_ref[pl.ds(col0, ck), :],
                       preferred_element_type=jnp.float32)

    def dot_gmask():
        rows = col0 + jax.lax.broadcasted_iota(
            jnp.int32, (ck, o_ref.shape[1]), 0)
        gc = jnp.where(rows >= start, g_ref[pl.ds(col0, ck), :], 0.0)
        return jnp.dot(adj_ref[...], gc, preferred_element_type=jnp.float32)

    def dot_edge():
        rows = col0 + jax.lax.broadcasted_iota(
            jnp.int32, (ck, o_ref.shape[1]), 0)
        gc = jnp.where((rows >= start) & (rows < n),
                       g_ref[pl.ds(col0, ck), :], 0.0)
        cols = col0 + jax.lax.broadcasted_iota(jnp.int32, adj_ref.shape, 1)
        a = jnp.where(cols < n, adj_ref[...], 0.0)
        return jnp.dot(a, gc, preferred_element_type=jnp.float32)

    contrib = jax.lax.cond(
        k == nkc - 1, dot_edge,
        lambda: jax.lax.cond(col0 >= start, dot_plain, dot_gmask))

    @pl.when(sf[t] == 1)
    def _first():
        o_ref[...] = contrib

    @pl.when(sf[t] == 0)
    def _accum():
        o_ref[...] += contrib

    @pl.when(sl[t] == 1)
    def _epilogue():
        o_ref[...] = jnp.maximum(d_ref[...] * (o_ref[...] + y1_ref[...]), 0.0)


def kernel(x, adj, W):
    n, f_in = x.shape
    f_out = W.shape[0]

    bm = _block(n, 400)
    ni = n // bm
    ck = min(1280, ((n + 127) // 128) * 128)
    nkc = -(-n // ck)
    gpad = nkc * ck

    d, g, y1 = pl.pallas_call(
        functools.partial(_fwd_body, bm),
        grid=(ni,),
        in_specs=[
            pl.BlockSpec((bm, n), lambda i: (i, 0)),
            pl.BlockSpec((bm, f_in), lambda i: (i, 0)),
            pl.BlockSpec((f_out, f_in), lambda i: (0, 0)),
        ],
        out_specs=[
            pl.BlockSpec((bm, 1), lambda i: (i, 0)),
            pl.BlockSpec((bm, f_out), lambda i: (i, 0)),
            pl.BlockSpec((bm, f_out), lambda i: (i, 0)),
        ],
        out_shape=[
            jax.ShapeDtypeStruct((n, 1), jnp.float32),
            jax.ShapeDtypeStruct((gpad, f_out), jnp.float32),
            jax.ShapeDtypeStruct((n, f_out), jnp.float32),
        ],
        scratch_shapes=[pltpu.VMEM((n, f_out), jnp.float32)],
    )(adj, x, W)

    i_l, k_l, f_l, l_l = [], [], [], []
    for i in range(ni):
        kmin = min((bm * (i + 1)) // ck, nkc - 1)
        for k in range(kmin, nkc):
            i_l.append(i)
            k_l.append(k)
            f_l.append(1 if k == kmin else 0)
            l_l.append(1 if k == nkc - 1 else 0)

    grid_spec = pltpu.PrefetchScalarGridSpec(
        num_scalar_prefetch=4,
        grid=(len(i_l),),
        in_specs=[
            pl.BlockSpec((bm, ck), lambda t, si, sk, sf, sl: (si[t], sk[t])),
            pl.BlockSpec((gpad, f_out), lambda t, si, sk, sf, sl: (0, 0)),
            pl.BlockSpec((bm, f_out), lambda t, si, sk, sf, sl: (si[t], 0)),
            pl.BlockSpec((bm, 1), lambda t, si, sk, sf, sl: (si[t], 0)),
        ],
        out_specs=pl.BlockSpec(
            (bm, f_out), lambda t, si, sk, sf, sl: (si[t], 0)),
    )

    out = pl.pallas_call(
        functools.partial(_upper_body, bm, ck, nkc, n),
        grid_spec=grid_spec,
        out_shape=jax.ShapeDtypeStruct((n, f_out), jnp.float32),
    )(jnp.asarray(np.asarray(i_l), jnp.int32),
      jnp.asarray(np.asarray(k_l), jnp.int32),
      jnp.asarray(np.asarray(f_l), jnp.int32),
      jnp.asarray(np.asarray(l_l), jnp.int32),
      adj, g, y1, d)

    return out


# speedup vs baseline: 1.1099x; 1.0904x over previous
import functools

import jax
import jax.numpy as jnp
import numpy as np
from jax.experimental import pallas as pl
from jax.experimental.pallas import tpu as pltpu

_EPS = 1e-08


def _block(n: int, target: int) -> int:
    for b in range(min(n, target), 7, -1):
        if n % b == 0 and b % 8 == 0:
            return b
    return n


def _fwd_body(bm, adj_ref, x_ref, w_ref, d_ref, g_ref, y1_ref, gs_ref):
    i = pl.program_id(0)

    @pl.when(i == 0)
    def _init():
        gs_ref[...] = jnp.zeros_like(gs_ref)

    a = adj_ref[...]
    deg = jnp.sum(a, axis=1, keepdims=True) + 1.0
    dis = jax.lax.rsqrt(jnp.maximum(deg, _EPS))
    d_ref[...] = dis
    h = jax.lax.dot_general(
        x_ref[...], w_ref[...], (((1,), (1,)), ((), ())),
        preferred_element_type=jnp.float32)
    g = dis * h
    g_ref[...] = g
    gs_ref[pl.ds(i * bm, bm), :] = g
    y1_ref[...] = g + jnp.dot(a, gs_ref[...],
                              preferred_element_type=jnp.float32)


def _upper_body(bm, ck, nkc, n,
                si, sk, sf, sl, adj_ref, g_ref, y1_ref, d_ref, o_ref):
    t = pl.program_id(0)
    i = si[t]
    k = sk[t]
    col0 = k * ck
    start = (i + 1) * bm

    def dot_plain():
        return jnp.dot(adj_ref[...], g_ref[pl.ds(col0, ck), :],
                       preferred_element_type=jnp.float32)

    def dot_gmask():
        rows = col0 + jax.lax.broadcasted_iota(
            jnp.int32, (ck, o_ref.shape[1]), 0)
        gc = jnp.where(rows >= start, g_ref[pl.ds(col0, ck), :], 0.0)
        return jnp.dot(adj_ref[...], gc, preferred_element_type=jnp.float32)

    def dot_edge():
        rows = col0 + jax.lax.broadcasted_iota(
            jnp.int32, (ck, o_ref.shape[1]), 0)
        gc = jnp.where((rows >= start) & (rows < n),
                       g_ref[pl.ds(col0, ck), :], 0.0)
        cols = col0 + jax.lax.broadcasted_iota(jnp.int32, adj_ref.shape, 1)
        a = jnp.where(cols < n, adj_ref[...], 0.0)
        return jnp.dot(a, gc, preferred_element_type=jnp.float32)

    contrib = jax.lax.cond(
        k == nkc - 1, dot_edge,
        lambda: jax.lax.cond(col0 >= start, dot_plain, dot_gmask))

    @pl.when(sf[t] == 1)
    def _first():
        o_ref[...] = contrib

    @pl.when(sf[t] == 0)
    def _accum():
        o_ref[...] += contrib

    @pl.when(sl[t] == 1)
    def _epilogue():
        o_ref[...] = jnp.maximum(d_ref[...] * (o_ref[...] + y1_ref[...]), 0.0)


def kernel(x, adj, W):
    n, f_in = x.shape
    f_out = W.shape[0]

    bm = _block(n, 400)
    ni = n // bm
    ck = min(2048, ((n + 127) // 128) * 128)
    nkc = -(-n // ck)
    gpad = nkc * ck

    d, g, y1 = pl.pallas_call(
        functools.partial(_fwd_body, bm),
        grid=(ni,),
        in_specs=[
            pl.BlockSpec((bm, n), lambda i: (i, 0)),
            pl.BlockSpec((bm, f_in), lambda i: (i, 0)),
            pl.BlockSpec((f_out, f_in), lambda i: (0, 0)),
        ],
        out_specs=[
            pl.BlockSpec((bm, 1), lambda i: (i, 0)),
            pl.BlockSpec((bm, f_out), lambda i: (i, 0)),
            pl.BlockSpec((bm, f_out), lambda i: (i, 0)),
        ],
        out_shape=[
            jax.ShapeDtypeStruct((n, 1), jnp.float32),
            jax.ShapeDtypeStruct((gpad, f_out), jnp.float32),
            jax.ShapeDtypeStruct((n, f_out), jnp.float32),
        ],
        scratch_shapes=[pltpu.VMEM((n, f_out), jnp.float32)],
    )(adj, x, W)

    i_l, k_l, f_l, l_l = [], [], [], []
    for i in range(ni):
        kmin = min((bm * (i + 1)) // ck, nkc - 1)
        for k in range(kmin, nkc):
            i_l.append(i)
            k_l.append(k)
            f_l.append(1 if k == kmin else 0)
            l_l.append(1 if k == nkc - 1 else 0)

    grid_spec = pltpu.PrefetchScalarGridSpec(
        num_scalar_prefetch=4,
        grid=(len(i_l),),
        in_specs=[
            pl.BlockSpec((bm, ck), lambda t, si, sk, sf, sl: (si[t], sk[t])),
            pl.BlockSpec((gpad, f_out), lambda t, si, sk, sf, sl: (0, 0)),
            pl.BlockSpec((bm, f_out), lambda t, si, sk, sf, sl: (si[t], 0)),
            pl.BlockSpec((bm, 1), lambda t, si, sk, sf, sl: (si[t], 0)),
        ],
        out_specs=pl.BlockSpec(
            (bm, f_out), lambda t, si, sk, sf, sl: (si[t], 0)),
    )

    out = pl.pallas_call(
        functools.partial(_upper_body, bm, ck, nkc, n),
        grid_spec=grid_spec,
        out_shape=jax.ShapeDtypeStruct((n, f_out), jnp.float32),
    )(jnp.asarray(np.asarray(i_l), jnp.int32),
      jnp.asarray(np.asarray(k_l), jnp.int32),
      jnp.asarray(np.asarray(f_l), jnp.int32),
      jnp.asarray(np.asarray(l_l), jnp.int32),
      adj, g, y1, d)

    return out


# speedup vs baseline: 1.2483x; 1.1246x over previous
import functools

import jax
import jax.numpy as jnp
import numpy as np
from jax.experimental import pallas as pl
from jax.experimental.pallas import tpu as pltpu

_EPS = 1e-08


def _block(n: int, target: int) -> int:
    for b in range(min(n, target), 7, -1):
        if n % b == 0 and b % 8 == 0:
            return b
    return n


def _fwd_body(bm, adj_ref, x_ref, w_ref, d_ref, g_ref, y1_ref, gs_ref):
    i = pl.program_id(0)

    @pl.when(i == 0)
    def _init():
        gs_ref[...] = jnp.zeros_like(gs_ref)

    a = adj_ref[...]
    deg = jnp.sum(a, axis=1, keepdims=True) + 1.0
    dis = jax.lax.rsqrt(jnp.maximum(deg, _EPS))
    d_ref[...] = dis
    h = jax.lax.dot_general(
        x_ref[...], w_ref[...], (((1,), (1,)), ((), ())),
        preferred_element_type=jnp.float32)
    g = dis * h
    g_ref[...] = g
    gs_ref[pl.ds(i * bm, bm), :] = g
    y1_ref[...] = g + jnp.dot(a, gs_ref[...],
                              preferred_element_type=jnp.float32)


def _upper_body(bm, ck, nkc, n,
                si, sk, sf, sl, adj_ref, g_ref, y1_ref, d_ref, o_ref):
    t = pl.program_id(0)
    i = si[t]
    k = sk[t]
    col0 = k * ck
    start = (i + 1) * bm

    def dot_plain():
        return jnp.dot(adj_ref[...], g_ref[pl.ds(col0, ck), :],
                       preferred_element_type=jnp.float32)

    def dot_gmask():
        rows = col0 + jax.lax.broadcasted_iota(
            jnp.int32, (ck, o_ref.shape[1]), 0)
        gc = jnp.where(rows >= start, g_ref[pl.ds(col0, ck), :], 0.0)
        return jnp.dot(adj_ref[...], gc, preferred_element_type=jnp.float32)

    def dot_edge():
        rows = col0 + jax.lax.broadcasted_iota(
            jnp.int32, (ck, o_ref.shape[1]), 0)
        gc = jnp.where((rows >= start) & (rows < n),
                       g_ref[pl.ds(col0, ck), :], 0.0)
        cols = col0 + jax.lax.broadcasted_iota(jnp.int32, adj_ref.shape, 1)
        a = jnp.where(cols < n, adj_ref[...], 0.0)
        return jnp.dot(a, gc, preferred_element_type=jnp.float32)

    contrib = jax.lax.cond(
        k == nkc - 1, dot_edge,
        lambda: jax.lax.cond(col0 >= start, dot_plain, dot_gmask))

    @pl.when(sf[t] == 1)
    def _first():
        o_ref[...] = contrib

    @pl.when(sf[t] == 0)
    def _accum():
        o_ref[...] += contrib

    @pl.when(sl[t] == 1)
    def _epilogue():
        o_ref[...] = jnp.maximum(d_ref[...] * (o_ref[...] + y1_ref[...]), 0.0)


def kernel(x, adj, W):
    n, f_in = x.shape
    f_out = W.shape[0]

    bm = _block(n, 400)
    ni = n // bm
    bm2 = _block(n, 1000)
    ck = min(2048, ((n + 127) // 128) * 128)
    nkc = -(-n // ck)
    gpad = nkc * ck

    d, g, y1 = pl.pallas_call(
        functools.partial(_fwd_body, bm),
        grid=(ni,),
        in_specs=[
            pl.BlockSpec((bm, n), lambda i: (i, 0)),
            pl.BlockSpec((bm, f_in), lambda i: (i, 0)),
            pl.BlockSpec((f_out, f_in), lambda i: (0, 0)),
        ],
        out_specs=[
            pl.BlockSpec((bm, 1), lambda i: (i, 0)),
            pl.BlockSpec((bm, f_out), lambda i: (i, 0)),
            pl.BlockSpec((bm, f_out), lambda i: (i, 0)),
        ],
        out_shape=[
            jax.ShapeDtypeStruct((n, 1), jnp.float32),
            jax.ShapeDtypeStruct((gpad, f_out), jnp.float32),
            jax.ShapeDtypeStruct((n, f_out), jnp.float32),
        ],
        scratch_shapes=[pltpu.VMEM((n, f_out), jnp.float32)],
    )(adj, x, W)

    i_l, k_l, f_l, l_l = [], [], [], []
    for i in range(n // bm2):
        kmin = min((bm2 * (i + 1)) // ck, nkc - 1)
        for k in range(kmin, nkc):
            i_l.append(i)
            k_l.append(k)
            f_l.append(1 if k == kmin else 0)
            l_l.append(1 if k == nkc - 1 else 0)

    grid_spec = pltpu.PrefetchScalarGridSpec(
        num_scalar_prefetch=4,
        grid=(len(i_l),),
        in_specs=[
            pl.BlockSpec((bm2, ck), lambda t, si, sk, sf, sl: (si[t], sk[t])),
            pl.BlockSpec((gpad, f_out), lambda t, si, sk, sf, sl: (0, 0)),
            pl.BlockSpec((bm2, f_out), lambda t, si, sk, sf, sl: (si[t], 0)),
            pl.BlockSpec((bm2, 1), lambda t, si, sk, sf, sl: (si[t], 0)),
        ],
        out_specs=pl.BlockSpec(
            (bm2, f_out), lambda t, si, sk, sf, sl: (si[t], 0)),
    )

    out = pl.pallas_call(
        functools.partial(_upper_body, bm2, ck, nkc, n),
        grid_spec=grid_spec,
        out_shape=jax.ShapeDtypeStruct((n, f_out), jnp.float32),
    )(jnp.asarray(np.asarray(i_l), jnp.int32),
      jnp.asarray(np.asarray(k_l), jnp.int32),
      jnp.asarray(np.asarray(f_l), jnp.int32),
      jnp.asarray(np.asarray(l_l), jnp.int32),
      adj, g, y1, d)

    return out


# speedup vs baseline: 1.2517x; 1.0027x over previous
import functools

import jax
import jax.numpy as jnp
import numpy as np
from jax.experimental import pallas as pl
from jax.experimental.pallas import tpu as pltpu

_EPS = 1e-08


def _fwd_body(bm, ck, adj_ref, x_ref, w_ref, d_ref, g_ref, y1_ref,
              gs_ref, stage_ref):
    i = pl.program_id(0)

    @pl.when(i == 0)
    def _init():
        gs_ref[...] = jnp.zeros_like(gs_ref)

    spg = ck // bm

    @pl.when(jnp.logical_and(i % spg == 0, i > 0))
    def _flush():
        gs_ref[pl.ds((i // spg - 1) * ck, ck), :] = stage_ref[...]

    a = adj_ref[...]
    deg = jnp.sum(a, axis=1, keepdims=True) + 1.0
    dis = jax.lax.rsqrt(jnp.maximum(deg, _EPS))
    d_ref[...] = dis
    h = jax.lax.dot_general(
        x_ref[...], w_ref[...], (((1,), (1,)), ((), ())),
        preferred_element_type=jnp.float32)
    g = dis * h
    g_ref[...] = g
    stage_ref[pl.ds((i % spg) * bm, bm), :] = g
    y1_ref[...] = g + jnp.dot(a, gs_ref[...],
                              preferred_element_type=jnp.float32)


def _upper_body(ck, nkc, n,
                si, sk, sf, sl, adj_ref, g_ref, y1_ref, d_ref, o_ref):
    t = pl.program_id(0)
    k = sk[t]
    col0 = k * ck

    def dot_plain():
        return jnp.dot(adj_ref[...], g_ref[pl.ds(col0, ck), :],
                       preferred_element_type=jnp.float32)

    def dot_edge():
        rows = col0 + jax.lax.broadcasted_iota(
            jnp.int32, (ck, o_ref.shape[1]), 0)
        gc = jnp.where(rows < n, g_ref[pl.ds(col0, ck), :], 0.0)
        cols = col0 + jax.lax.broadcasted_iota(jnp.int32, adj_ref.shape, 1)
        a = jnp.where(cols < n, adj_ref[...], 0.0)
        return jnp.dot(a, gc, preferred_element_type=jnp.float32)

    contrib = jax.lax.cond(k == nkc - 1, dot_edge, dot_plain)

    @pl.when(sf[t] == 1)
    def _first():
        o_ref[...] = contrib

    @pl.when(sf[t] == 0)
    def _accum():
        o_ref[...] += contrib

    @pl.when(sl[t] == 1)
    def _epilogue():
        o_ref[...] = jnp.maximum(d_ref[...] * (o_ref[...] + y1_ref[...]), 0.0)


def kernel(x, adj, W):
    n, f_in = x.shape
    f_out = W.shape[0]

    ck = min(2048, ((n + 127) // 128) * 128)
    bm = ck // 4
    ni = -(-n // bm)
    nkc = -(-n // ck)
    gpad = nkc * ck

    d, g, y1 = pl.pallas_call(
        functools.partial(_fwd_body, bm, ck),
        grid=(ni,),
        in_specs=[
            pl.BlockSpec((bm, n), lambda i: (i, 0)),
            pl.BlockSpec((bm, f_in), lambda i: (i, 0)),
            pl.BlockSpec((f_out, f_in), lambda i: (0, 0)),
        ],
        out_specs=[
            pl.BlockSpec((bm, 1), lambda i: (i, 0)),
            pl.BlockSpec((bm, f_out), lambda i: (i, 0)),
            pl.BlockSpec((bm, f_out), lambda i: (i, 0)),
        ],
        out_shape=[
            jax.ShapeDtypeStruct((n, 1), jnp.float32),
            jax.ShapeDtypeStruct((gpad, f_out), jnp.float32),
            jax.ShapeDtypeStruct((n, f_out), jnp.float32),
        ],
        scratch_shapes=[
            pltpu.VMEM((n, f_out), jnp.float32),
            pltpu.VMEM((ck, f_out), jnp.float32),
        ],
    )(adj, x, W)

    i_l, k_l, f_l, l_l = [], [], [], []
    for gi in range(nkc):
        for k in range(gi, nkc):
            i_l.append(gi)
            k_l.append(k)
            f_l.append(1 if k == gi else 0)
            l_l.append(1 if k == nkc - 1 else 0)

    grid_spec = pltpu.PrefetchScalarGridSpec(
        num_scalar_prefetch=4,
        grid=(len(i_l),),
        in_specs=[
            pl.BlockSpec((ck, ck), lambda t, si, sk, sf, sl: (si[t], sk[t])),
            pl.BlockSpec((gpad, f_out), lambda t, si, sk, sf, sl: (0, 0)),
            pl.BlockSpec((ck, f_out), lambda t, si, sk, sf, sl: (si[t], 0)),
            pl.BlockSpec((ck, 1), lambda t, si, sk, sf, sl: (si[t], 0)),
        ],
        out_specs=pl.BlockSpec(
            (ck, f_out), lambda t, si, sk, sf, sl: (si[t], 0)),
    )

    out = pl.pallas_call(
        functools.partial(_upper_body, ck, nkc, n),
        grid_spec=grid_spec,
        out_shape=jax.ShapeDtypeStruct((n, f_out), jnp.float32),
    )(jnp.asarray(np.asarray(i_l), jnp.int32),
      jnp.asarray(np.asarray(k_l), jnp.int32),
      jnp.asarray(np.asarray(f_l), jnp.int32),
      jnp.asarray(np.asarray(l_l), jnp.int32),
      adj, g, y1, d)

    return out


# speedup vs baseline: 1.2731x; 1.0171x over previous
import functools

import jax
import jax.numpy as jnp
import numpy as np
from jax.experimental import pallas as pl
from jax.experimental.pallas import tpu as pltpu

_EPS = 1e-08


def _fwd_body(bm, ck, nkc, ni, adj_ref, x_ref, w_ref, d_ref, g_ref,
              y1_ref, y2_ref, gs_ref, stage_ref, stash_ref):
    i = pl.program_id(0)

    @pl.when(i == 0)
    def _init():
        gs_ref[...] = jnp.zeros_like(gs_ref)

    spg = ck // bm
    grp = i // spg

    @pl.when(jnp.logical_and(i % spg == 0, i > 0))
    def _flush():
        gs_ref[pl.ds((grp - 1) * ck, ck), :] = stage_ref[...]

    a = adj_ref[...]
    deg = jnp.sum(a, axis=1, keepdims=True) + 1.0
    dis = jax.lax.rsqrt(jnp.maximum(deg, _EPS))
    d_ref[...] = dis
    h = jax.lax.dot_general(
        x_ref[...], w_ref[...], (((1,), (1,)), ((), ())),
        preferred_element_type=jnp.float32)
    g = dis * h
    g_ref[...] = g
    stage_ref[pl.ds((i % spg) * bm, bm), :] = g
    y1_ref[...] = g + jnp.dot(a, gs_ref[...],
                              preferred_element_type=jnp.float32)

    if nkc < 2:
        @pl.when(i == ni - 1)
        def _last0():
            y2_ref[...] = jnp.zeros_like(y2_ref)
        return

    for s in range(nkc - 1):
        @pl.when(grp == s)
        def _stash(s=s):
            stash_ref[pl.ds((i % spg) * bm, bm), :] = (
                adj_ref[:, s * ck:(s + 1) * ck].astype(jnp.bfloat16))

    @pl.when(jnp.logical_and(i % spg == spg - 1, grp < nkc - 1))
    def _diag():
        y2_ref[...] = jnp.dot(
            stash_ref[...], stage_ref[...].astype(jnp.bfloat16),
            preferred_element_type=jnp.float32)

    @pl.when(i == ni - 1)
    def _last():
        y2_ref[...] = jnp.zeros_like(y2_ref)


def _upper_body(ck, nkc, n,
                si, sk, sf, sl, adj_ref, g_ref, y1_ref, y2_ref, d_ref,
                o_ref):
    t = pl.program_id(0)
    k = sk[t]
    col0 = k * ck

    def dot_plain():
        return jnp.dot(adj_ref[...], g_ref[pl.ds(col0, ck), :],
                       preferred_element_type=jnp.float32)

    def dot_edge():
        rows = col0 + jax.lax.broadcasted_iota(
            jnp.int32, (ck, o_ref.shape[1]), 0)
        gc = jnp.where(rows < n, g_ref[pl.ds(col0, ck), :], 0.0)
        cols = col0 + jax.lax.broadcasted_iota(jnp.int32, adj_ref.shape, 1)
        a = jnp.where(cols < n, adj_ref[...], 0.0)
        return jnp.dot(a, gc, preferred_element_type=jnp.float32)

    contrib = jax.lax.cond(k == nkc - 1, dot_edge, dot_plain)

    @pl.when(sf[t] == 1)
    def _first():
        o_ref[...] = contrib

    @pl.when(sf[t] == 0)
    def _accum():
        o_ref[...] += contrib

    @pl.when(sl[t] == 1)
    def _epilogue():
        o_ref[...] = jnp.maximum(
            d_ref[...] * (o_ref[...] + y1_ref[...] + y2_ref[...]), 0.0)


def kernel(x, adj, W):
    n, f_in = x.shape
    f_out = W.shape[0]

    ck = min(2048, ((n + 127) // 128) * 128)
    bm = ck // 8
    ni = -(-n // bm)
    nkc = -(-n // ck)
    gpad = nkc * ck

    d, g, y1, y2 = pl.pallas_call(
        functools.partial(_fwd_body, bm, ck, nkc, ni),
        grid=(ni,),
        in_specs=[
            pl.BlockSpec((bm, n), lambda i: (i, 0)),
            pl.BlockSpec((bm, f_in), lambda i: (i, 0)),
            pl.BlockSpec((f_out, f_in), lambda i: (0, 0)),
        ],
        out_specs=[
            pl.BlockSpec((bm, 1), lambda i: (i, 0)),
            pl.BlockSpec((bm, f_out), lambda i: (i, 0)),
            pl.BlockSpec((bm, f_out), lambda i: (i, 0)),
            pl.BlockSpec((ck, f_out), lambda i: (i // (ck // bm), 0)),
        ],
        out_shape=[
            jax.ShapeDtypeStruct((n, 1), jnp.float32),
            jax.ShapeDtypeStruct((gpad, f_out), jnp.float32),
            jax.ShapeDtypeStruct((n, f_out), jnp.float32),
            jax.ShapeDtypeStruct((n, f_out), jnp.float32),
        ],
        scratch_shapes=[
            pltpu.VMEM((n, f_out), jnp.float32),
            pltpu.VMEM((ck, f_out), jnp.float32),
            pltpu.VMEM((ck, ck), jnp.bfloat16),
        ],
    )(adj, x, W)

    i_l, k_l, f_l, l_l = [], [], [], []
    for gi in range(nkc):
        ks = list(range(gi + 1, nkc)) if gi < nkc - 1 else [nkc - 1]
        for k in ks:
            i_l.append(gi)
            k_l.append(k)
            f_l.append(1 if k == ks[0] else 0)
            l_l.append(1 if k == ks[-1] else 0)

    grid_spec = pltpu.PrefetchScalarGridSpec(
        num_scalar_prefetch=4,
        grid=(len(i_l),),
        in_specs=[
            pl.BlockSpec((ck, ck), lambda t, si, sk, sf, sl: (si[t], sk[t])),
            pl.BlockSpec((gpad, f_out), lambda t, si, sk, sf, sl: (0, 0)),
            pl.BlockSpec((ck, f_out), lambda t, si, sk, sf, sl: (si[t], 0)),
            pl.BlockSpec((ck, f_out), lambda t, si, sk, sf, sl: (si[t], 0)),
            pl.BlockSpec((ck, 1), lambda t, si, sk, sf, sl: (si[t], 0)),
        ],
        out_specs=pl.BlockSpec(
            (ck, f_out), lambda t, si, sk, sf, sl: (si[t], 0)),
    )

    out = pl.pallas_call(
        functools.partial(_upper_body, ck, nkc, n),
        grid_spec=grid_spec,
        out_shape=jax.ShapeDtypeStruct((n, f_out), jnp.float32),
    )(jnp.asarray(np.asarray(i_l), jnp.int32),
      jnp.asarray(np.asarray(k_l), jnp.int32),
      jnp.asarray(np.asarray(f_l), jnp.int32),
      jnp.asarray(np.asarray(l_l), jnp.int32),
      adj, g, y1, y2, d)

    return out


# speedup vs baseline: 1.2948x; 1.0170x over previous
import functools

import jax
import jax.numpy as jnp
import numpy as np
from jax.experimental import pallas as pl
from jax.experimental.pallas import tpu as pltpu

_EPS = 1e-08


def _fwd_body(bm, ck, nkc, ni, fo, adj_ref, x_ref, w_ref, d_ref, g_ref,
              y1_ref, y2_ref, gs_ref, stage_ref, stash_ref):
    i = pl.program_id(0)
    fp = fo + 8

    @pl.when(i == 0)
    def _init():
        lane = jax.lax.broadcasted_iota(jnp.int32, gs_ref.shape, 1)
        gs_ref[...] = jnp.where(lane >= fo, 1.0, 0.0)
        stage_ref[:, fo:] = jnp.ones_like(stage_ref[:, fo:])

    spg = ck // bm
    grp = i // spg

    @pl.when(jnp.logical_and(i % spg == 0, i > 0))
    def _flush():
        gs_ref[pl.ds((grp - 1) * ck, ck), :] = stage_ref[...]

    a = adj_ref[...]
    y1full = jnp.dot(a, gs_ref[...], preferred_element_type=jnp.float32)
    deg = y1full[:, fo:fo + 1] + 1.0
    dis = jax.lax.rsqrt(jnp.maximum(deg, _EPS))
    d_ref[...] = dis
    h = jax.lax.dot_general(
        x_ref[...], w_ref[...], (((1,), (1,)), ((), ())),
        preferred_element_type=jnp.float32)
    g = dis * h
    g_ref[...] = g
    stage_ref[pl.ds((i % spg) * bm, bm), :fo] = g
    y1_ref[...] = g + y1full[:, :fo]

    if nkc < 2:
        @pl.when(i == ni - 1)
        def _last0():
            y2_ref[...] = jnp.zeros_like(y2_ref)
        return

    for s in range(nkc - 1):
        @pl.when(grp == s)
        def _stash(s=s):
            stash_ref[pl.ds((i % spg) * bm, bm), :] = (
                adj_ref[:, s * ck:(s + 1) * ck].astype(jnp.bfloat16))

    @pl.when(jnp.logical_and(i % spg == spg - 1, grp < nkc - 1))
    def _diag():
        y2_ref[...] = jnp.dot(
            stash_ref[...], stage_ref[:, :fo].astype(jnp.bfloat16),
            preferred_element_type=jnp.float32)

    @pl.when(i == ni - 1)
    def _last():
        y2_ref[...] = jnp.zeros_like(y2_ref)


def _upper_body(ck, nkc, n,
                si, sk, sf, sl, adj_ref, g_ref, y1_ref, y2_ref, d_ref,
                o_ref):
    t = pl.program_id(0)
    k = sk[t]
    col0 = k * ck

    def dot_plain():
        return jnp.dot(adj_ref[...], g_ref[pl.ds(col0, ck), :],
                       preferred_element_type=jnp.float32)

    def dot_edge():
        rows = col0 + jax.lax.broadcasted_iota(
            jnp.int32, (ck, o_ref.shape[1]), 0)
        gc = jnp.where(rows < n, g_ref[pl.ds(col0, ck), :], 0.0)
        cols = col0 + jax.lax.broadcasted_iota(jnp.int32, adj_ref.shape, 1)
        a = jnp.where(cols < n, adj_ref[...], 0.0)
        return jnp.dot(a, gc, preferred_element_type=jnp.float32)

    contrib = jax.lax.cond(k == nkc - 1, dot_edge, dot_plain)

    @pl.when(sf[t] == 1)
    def _first():
        o_ref[...] = contrib

    @pl.when(sf[t] == 0)
    def _accum():
        o_ref[...] += contrib

    @pl.when(sl[t] == 1)
    def _epilogue():
        o_ref[...] = jnp.maximum(
            d_ref[...] * (o_ref[...] + y1_ref[...] + y2_ref[...]), 0.0)


def kernel(x, adj, W):
    n, f_in = x.shape
    f_out = W.shape[0]

    ck = min(2048, ((n + 127) // 128) * 128)
    bm = ck // 8
    ni = -(-n // bm)
    nkc = -(-n // ck)
    gpad = nkc * ck

    d, g, y1, y2 = pl.pallas_call(
        functools.partial(_fwd_body, bm, ck, nkc, ni, f_out),
        grid=(ni,),
        in_specs=[
            pl.BlockSpec((bm, n), lambda i: (i, 0)),
            pl.BlockSpec((bm, f_in), lambda i: (i, 0)),
            pl.BlockSpec((f_out, f_in), lambda i: (0, 0)),
        ],
        out_specs=[
            pl.BlockSpec((bm, 1), lambda i: (i, 0)),
            pl.BlockSpec((bm, f_out), lambda i: (i, 0)),
            pl.BlockSpec((bm, f_out), lambda i: (i, 0)),
            pl.BlockSpec((ck, f_out), lambda i: (i // (ck // bm), 0)),
        ],
        out_shape=[
            jax.ShapeDtypeStruct((n, 1), jnp.float32),
            jax.ShapeDtypeStruct((gpad, f_out), jnp.float32),
            jax.ShapeDtypeStruct((n, f_out), jnp.float32),
            jax.ShapeDtypeStruct((n, f_out), jnp.float32),
        ],
        scratch_shapes=[
            pltpu.VMEM((n, f_out + 8), jnp.float32),
            pltpu.VMEM((ck, f_out + 8), jnp.float32),
            pltpu.VMEM((ck, ck), jnp.bfloat16),
        ],
    )(adj, x, W)

    i_l, k_l, f_l, l_l = [], [], [], []
    for gi in range(nkc):
        ks = list(range(gi + 1, nkc)) if gi < nkc - 1 else [nkc - 1]
        for k in ks:
            i_l.append(gi)
            k_l.append(k)
            f_l.append(1 if k == ks[0] else 0)
            l_l.append(1 if k == ks[-1] else 0)

    grid_spec = pltpu.PrefetchScalarGridSpec(
        num_scalar_prefetch=4,
        grid=(len(i_l),),
        in_specs=[
            pl.BlockSpec((ck, ck), lambda t, si, sk, sf, sl: (si[t], sk[t])),
            pl.BlockSpec((gpad, f_out), lambda t, si, sk, sf, sl: (0, 0)),
            pl.BlockSpec((ck, f_out), lambda t, si, sk, sf, sl: (si[t], 0)),
            pl.BlockSpec((ck, f_out), lambda t, si, sk, sf, sl: (si[t], 0)),
            pl.BlockSpec((ck, 1), lambda t, si, sk, sf, sl: (si[t], 0)),
        ],
        out_specs=pl.BlockSpec(
            (ck, f_out), lambda t, si, sk, sf, sl: (si[t], 0)),
    )

    out = pl.pallas_call(
        functools.partial(_upper_body, ck, nkc, n),
        grid_spec=grid_spec,
        out_shape=jax.ShapeDtypeStruct((n, f_out), jnp.float32),
    )(jnp.asarray(np.asarray(i_l), jnp.int32),
      jnp.asarray(np.asarray(k_l), jnp.int32),
      jnp.asarray(np.asarray(f_l), jnp.int32),
      jnp.asarray(np.asarray(l_l), jnp.int32),
      adj, g, y1, y2, d)

    return out
